# Initial kernel scaffold; baseline (speedup 1.0000x reference)
import jax
import jax.numpy as jnp
from jax.experimental import pallas as pl


def kernel(x_article, x_tweet, x_user, h_article, h_tweet, h_user, c_article, c_tweet, c_user, batch_article, batch_tweet, batch_user, edge_index_posts, edge_index_mentions, edge_index_follows, edge_index_cites, params):
    raise NotImplementedError("write your pallas kernel here")


# speedup vs baseline: 1.2609x; 1.2609x over previous
import functools

import jax
import jax.numpy as jnp
from jax import lax
from jax.experimental import pallas as pl
from jax.experimental.pallas import tpu as pltpu

_NT = ("article", "tweet", "user")
_SIZES = {"article": 10000, "tweet": 100000, "user": 50000}
_ETS = (("user", "posts", "tweet"), ("tweet", "mentions", "user"),
        ("user", "follows", "user"), ("tweet", "cites", "article"))
_IN_ETS = {"article": ("cites",), "tweet": ("posts",), "user": ("mentions", "follows")}
_SRC_OF = {"posts": "user", "mentions": "tweet", "follows": "user", "cites": "tweet"}
_D, _H, _OUT, _NB = 128, 64, 32, 64
_G = ("i", "f", "c", "o")
_BLK = 1024
_CNTW = 16


def _dense_kernel_body(k, nblk, *refs):
    x, h, c, b = refs[0], refs[1], refs[2], refs[3]
    sums = refs[4:4 + k]
    cnts = refs[4 + k:4 + 2 * k]
    Wx, Wr = refs[4 + 2 * k], refs[5 + 2 * k]
    Wls = refs[6 + 2 * k:6 + 3 * k]
    bias = refs[6 + 3 * k]
    h0o, c0o, pso, pco = refs[7 + 3 * k:11 + 3 * k]

    pre = jnp.dot(x[...], Wx[...], preferred_element_type=jnp.float32)
    pre = pre + jnp.dot(h[...], Wr[...], preferred_element_type=jnp.float32)
    for j in range(k):
        cnt = cnts[j][:, 0:1]
        mean = sums[j][...] * (1.0 / jnp.maximum(cnt, 1.0))
        pre = pre + jnp.dot(mean, Wls[j][...], preferred_element_type=jnp.float32)
    pre = pre + bias[...]

    ig = jax.nn.sigmoid(pre[:, 0:_H])
    fg = jax.nn.sigmoid(pre[:, _H:2 * _H])
    tg = jnp.tanh(pre[:, 2 * _H:3 * _H])
    og = jax.nn.sigmoid(pre[:, 3 * _H:4 * _H])
    c0 = fg * c[...] + ig * tg
    h0 = og * jnp.tanh(c0)
    h0o[...] = h0
    c0o[...] = c0

    hr = jnp.maximum(h0, 0.0)
    onehot = (b[...] == lax.broadcasted_iota(jnp.int32, (_BLK, _NB), 1)).astype(jnp.float32)
    ps_blk = lax.dot_general(onehot, hr, (((0,), (0,)), ((), ())),
                             preferred_element_type=jnp.float32)
    pc_blk = lax.dot_general(onehot, jnp.ones((_BLK, 8), jnp.float32),
                             (((0,), (0,)), ((), ())),
                             preferred_element_type=jnp.float32)

    i = pl.program_id(0)

    @pl.when(i == 0)
    def _():
        pso[...] = jnp.zeros_like(pso)
        pco[...] = jnp.zeros_like(pco)

    pso[...] += ps_blk
    pco[...] += pc_blk


def _dense_call(nt, x, h, c, batch, agg_sums, agg_cnts, Wx, Wr, Wls, bias):
    n = x.shape[0]
    k = len(agg_sums)
    nblk = -(-n // _BLK)
    npad = nblk * _BLK - n

    def padr(a):
        return jnp.pad(a, ((0, npad), (0, 0))) if npad else a

    xp, hp, cp = padr(x), padr(h), padr(c)
    bp = jnp.pad(batch, (0, npad), constant_values=_NB) if npad else batch
    bp = bp.reshape(nblk * _BLK, 1)
    sums_p = [padr(s) for s in agg_sums]
    cnts_p = [padr(cv) for cv in agg_cnts]

    row = lambda i: (i, 0)
    bcast = lambda i: (0, 0)
    in_specs = (
        [pl.BlockSpec((_BLK, _D), row), pl.BlockSpec((_BLK, _H), row),
         pl.BlockSpec((_BLK, _H), row), pl.BlockSpec((_BLK, 1), row)]
        + [pl.BlockSpec((_BLK, _H), row) for _ in range(k)]
        + [pl.BlockSpec((_BLK, _CNTW), row) for _ in range(k)]
        + [pl.BlockSpec((_D, 4 * _H), bcast), pl.BlockSpec((_H, 4 * _H), bcast)]
        + [pl.BlockSpec((_H, 4 * _H), bcast) for _ in range(k)]
        + [pl.BlockSpec((1, 4 * _H), bcast)]
    )
    out_specs = [
        pl.BlockSpec((_BLK, _H), row), pl.BlockSpec((_BLK, _H), row),
        pl.BlockSpec((_NB, _H), bcast), pl.BlockSpec((_NB, 8), bcast),
    ]
    out_shape = [
        jax.ShapeDtypeStruct((nblk * _BLK, _H), jnp.float32),
        jax.ShapeDtypeStruct((nblk * _BLK, _H), jnp.float32),
        jax.ShapeDtypeStruct((_NB, _H), jnp.float32),
        jax.ShapeDtypeStruct((_NB, 8), jnp.float32),
    ]
    h0, c0, ps, pc = pl.pallas_call(
        functools.partial(_dense_kernel_body, k, nblk),
        grid=(nblk,),
        in_specs=in_specs,
        out_specs=out_specs,
        out_shape=out_shape,
    )(xp, hp, cp, bp, *sums_p, *cnts_p, Wx, Wr, *Wls, bias)
    return h0[:n], c0[:n], ps, pc


def _final_kernel_body(psa, pca, pst, pct, psu, pcu, Wa, Wt, Wu, b, out):
    acc = b[...]
    for ps, pc, W in ((psa, pca, Wa), (pst, pct, Wt), (psu, pcu, Wu)):
        recip = 1.0 / jnp.maximum(pc[:, 0:1], 1.0)
        acc = acc + jnp.dot(ps[...] * recip, W[...], preferred_element_type=jnp.float32)
    out[...] = acc


def _final_call(pools, lin_W, lin_b):
    Wa, Wt, Wu = lin_W[:_H], lin_W[_H:2 * _H], lin_W[2 * _H:]
    args = []
    for nt in _NT:
        args += [pools[nt][0], pools[nt][1]]
    return pl.pallas_call(
        _final_kernel_body,
        out_shape=jax.ShapeDtypeStruct((_NB, _OUT), jnp.float32),
    )(*args, Wa, Wt, Wu, lin_b.reshape(1, _OUT))


def _aggregate(h_dict, ei_dict):
    sums, cnts = {}, {}
    for (s, r, d) in _ETS:
        ei = ei_dict[r]
        nd = _SIZES[d]
        msg = jnp.take(h_dict[s], ei[0], axis=0)
        sm = jax.ops.segment_sum(msg, ei[1], num_segments=nd)
        c1 = jax.ops.segment_sum(jnp.ones((ei.shape[1],), jnp.float32), ei[1],
                                 num_segments=nd)
        cw = jnp.zeros((nd, _CNTW), jnp.float32).at[:, 0].set(c1)
        sums[r] = sm
        cnts[r] = cw
    return sums, cnts


def kernel(x_article, x_tweet, x_user, h_article, h_tweet, h_user,
           c_article, c_tweet, c_user, batch_article, batch_tweet, batch_user,
           edge_index_posts, edge_index_mentions, edge_index_follows,
           edge_index_cites, params):
    x_dict = {"article": x_article, "tweet": x_tweet, "user": x_user}
    h_dict = {"article": h_article, "tweet": h_tweet, "user": h_user}
    c_dict = {"article": c_article, "tweet": c_tweet, "user": c_user}
    batch_dict = {"article": batch_article, "tweet": batch_tweet, "user": batch_user}
    ei_dict = {"posts": edge_index_posts, "mentions": edge_index_mentions,
               "follows": edge_index_follows, "cites": edge_index_cites}

    Wx = {nt: jnp.concatenate([params["W_%s_%s" % (g, nt)] for g in _G], axis=1)
          for nt in _NT}
    Wr = {nt: jnp.concatenate(
        [sum(params["Wr_%s_%s" % (g, r)] for r in _IN_ETS[nt]) for g in _G], axis=1)
        for nt in _NT}
    Wl = {r: jnp.concatenate([params["Wl_%s_%s" % (g, r)] for g in _G], axis=1)
          for r in ei_dict}
    bias = {nt: jnp.concatenate(
        [params["b_%s_%s" % (g, nt)]
         + sum(params["bl_%s_%s" % (g, r)] for r in _IN_ETS[nt])[None, :]
         for g in _G], axis=1)
        for nt in _NT}

    sums, cnts = _aggregate(h_dict, ei_dict)

    h0, c0, pools = {}, {}, {}
    for nt in _NT:
        rs = _IN_ETS[nt]
        h0[nt], c0[nt], ps, pc = _dense_call(
            nt, x_dict[nt], h_dict[nt], c_dict[nt], batch_dict[nt],
            [sums[r] for r in rs], [cnts[r] for r in rs],
            Wx[nt], Wr[nt], [Wl[r] for r in rs], bias[nt])
        pools[nt] = (ps, pc)

    out = _final_call(pools, params["lin_W"], params["lin_b"])
    return (out, h0["article"], h0["tweet"], h0["user"],
            c0["article"], c0["tweet"], c0["user"])


# speedup vs baseline: 1.7463x; 1.3850x over previous
import functools

import jax
import jax.numpy as jnp
from jax import lax
from jax.experimental import pallas as pl
from jax.experimental.pallas import tpu as pltpu
from jax.experimental.pallas import tpu_sc as plsc

_NT = ("article", "tweet", "user")
_SIZES = {"article": 10000, "tweet": 100000, "user": 50000}
_IN_ETS = {"article": ("cites",), "tweet": ("posts",), "user": ("mentions", "follows")}
_D, _H, _OUT, _NB = 128, 64, 32, 64
_G = ("i", "f", "c", "o")
_BLK = 1024
_CNTW = 16

_NC, _NS = 2, 16
_RND = 256
_SUB = 128
_CMAX = 20480
_DUMP = _CMAX

_ET_SPECS = (
    ("posts", "user", 200000, 49, (20480, 20480, 20480, 20480, 18432), 0),
    ("mentions", "tweet", 150000, 37, (20480, 20480, 9216), 5),
    ("follows", "user", 100000, 25, (20480, 20480, 9216), 8),
    ("cites", "tweet", 150000, 37, (10240,), 11),
)


def _steps(total, maxs):
    out, pos = [], 0
    while pos < total:
        out.append((pos, min(maxs, total - pos)))
        pos += min(maxs, total - pos)
    return out


def _sc_body(h_tweet, h_user, *refs):
    ins = refs[:8]
    outs = refs[8:16]
    (acc_s, acc_c, srcb, dstb, ldst, rows, ones, zc, sem) = refs[16:]
    tables = {"tweet": h_tweet, "user": h_user}

    cid = lax.axis_index("c")
    sid = lax.axis_index("s")

    zvec = jnp.zeros((16,), jnp.float32)
    onevec = jnp.where(lax.iota(jnp.int32, 16) == 0, 1.0, 0.0)

    def zcinit(j, carry):
        zc[j, :] = zvec
        return carry

    lax.fori_loop(0, _SUB, zcinit, 0)

    def oinit(j, carry):
        ones[j, :] = onevec
        return carry

    lax.fori_loop(0, _SUB, oinit, 0)

    dumpvec = jnp.full((16,), _DUMP, jnp.int32)

    for spec_i, (name, src_nt, E, nrnd, chunks, gbase) in enumerate(_ET_SPECS):
        table = tables[src_nt]
        src_ref, dst_ref = ins[2 * spec_i], ins[2 * spec_i + 1]
        sum_ref, cnt_ref = outs[2 * spec_i], outs[2 * spec_i + 1]
        slice_len = nrnd * _RND

        lo = 0
        for ch, C in enumerate(chunks):
            stripe = C // _NS

            @pl.when((gbase + ch) % 2 == cid)
            def _(src_ref=src_ref, dst_ref=dst_ref, sum_ref=sum_ref,
                  cnt_ref=cnt_ref, table=table, nrnd=nrnd, C=C, lo=lo,
                  slice_len=slice_len, stripe=stripe):
                def rz(j, carry):
                    for q in range(_H // 16):
                        rows[j, pl.ds(q * 16, 16)] = zvec
                    return carry

                lax.fori_loop(0, _RND, rz, 0)

                for (zp, zs) in _steps(stripe, _RND):
                    pltpu.sync_copy(rows.at[pl.ds(0, zs)],
                                    acc_s.at[pl.ds(sid * stripe + zp, zs)])
                for (zp, zs) in _steps(stripe, _SUB):
                    pltpu.sync_copy(zc.at[pl.ds(0, zs)],
                                    acc_c.at[pl.ds(sid * stripe + zp, zs)])

                plsc.subcore_barrier()

                ebase = sid * slice_len

                def rnd(r, carry):
                    eoff = ebase + r * _RND
                    pltpu.sync_copy(src_ref.at[pl.ds(eoff, _RND)], srcb)
                    pltpu.sync_copy(dst_ref.at[pl.ds(eoff, _RND)], dstb)

                    def vr(v, c2):
                        dv = dstb[pl.ds(v * 16, 16)]
                        m = (dv >= lo) & (dv < lo + C)
                        ldst[pl.ds(v * 16, 16)] = jnp.where(m, dv - lo, dumpvec)
                        return c2

                    lax.fori_loop(0, _RND // 16, vr, 0)

                    cps = [pltpu.async_copy(
                        table.at[srcb.at[pl.ds(j * _SUB, _SUB)]],
                        rows.at[pl.ds(j * _SUB, _SUB)], sem)
                        for j in range(_RND // _SUB)]
                    for cp in cps:
                        cp.wait()
                    cps = []
                    for j in range(_RND // _SUB):
                        cps.append(pltpu.async_copy(
                            rows.at[pl.ds(j * _SUB, _SUB)],
                            acc_s.at[ldst.at[pl.ds(j * _SUB, _SUB)]],
                            sem, add=True))
                        cps.append(pltpu.async_copy(
                            ones,
                            acc_c.at[ldst.at[pl.ds(j * _SUB, _SUB)]],
                            sem, add=True))
                    for cp in cps:
                        cp.wait()
                    return carry

                lax.fori_loop(0, nrnd, rnd, 0)

                plsc.subcore_barrier()

                for (zp, zs) in _steps(stripe, _RND):
                    pltpu.sync_copy(acc_s.at[pl.ds(sid * stripe + zp, zs)],
                                    sum_ref.at[pl.ds(lo + sid * stripe + zp, zs)])
                for (zp, zs) in _steps(stripe, _SUB):
                    pltpu.sync_copy(acc_c.at[pl.ds(sid * stripe + zp, zs)],
                                    cnt_ref.at[pl.ds(lo + sid * stripe + zp, zs)])

            lo += C


def _sc_aggregate(h_tweet, h_user, ei_dict):
    args = []
    for (name, src_nt, E, nrnd, chunks, gbase) in _ET_SPECS:
        ei = ei_dict[name]
        e_pad = _NS * nrnd * _RND
        pe = e_pad - ei.shape[1]
        args.append(jnp.pad(ei[0], (0, pe)))
        args.append(jnp.pad(ei[1], (0, pe), constant_values=-1))

    out_type = []
    for (name, src_nt, E, nrnd, chunks, gbase) in _ET_SPECS:
        nd_pad = sum(chunks)
        out_type.append(jax.ShapeDtypeStruct((nd_pad, _H), jnp.float32))
        out_type.append(jax.ShapeDtypeStruct((nd_pad, _CNTW), jnp.float32))

    scratch = [
        pltpu.VMEM_SHARED((_CMAX + 16, _H), jnp.float32),
        pltpu.VMEM_SHARED((_CMAX + 16, _CNTW), jnp.float32),
        pltpu.VMEM((_RND,), jnp.int32),
        pltpu.VMEM((_RND,), jnp.int32),
        pltpu.VMEM((_RND,), jnp.int32),
        pltpu.VMEM((_RND, _H), jnp.float32),
        pltpu.VMEM((_SUB, _CNTW), jnp.float32),
        pltpu.VMEM((_SUB, _CNTW), jnp.float32),
        pltpu.SemaphoreType.DMA,
    ]
    mesh = plsc.VectorSubcoreMesh(core_axis_name="c", subcore_axis_name="s")
    call = pl.kernel(_sc_body, out_type=out_type, mesh=mesh,
                     scratch_types=scratch,
                     compiler_params=pltpu.CompilerParams(
                         use_tc_tiling_on_sc=False))
    outs = call(h_tweet, h_user, *args)
    res = {}
    for i, (name, *_rest) in enumerate(_ET_SPECS):
        res[name] = (outs[2 * i], outs[2 * i + 1])
    return res


def _dense_kernel_body(k, n, *refs):
    x, h, c, b = refs[0], refs[1], refs[2], refs[3]
    sums = refs[4:4 + k]
    cnts = refs[4 + k:4 + 2 * k]
    Wx, Wr = refs[4 + 2 * k], refs[5 + 2 * k]
    Wls = refs[6 + 2 * k:6 + 3 * k]
    bias = refs[6 + 3 * k]
    h0o, c0o, pso, pco = refs[7 + 3 * k:11 + 3 * k]

    pre = jnp.dot(x[...], Wx[...], preferred_element_type=jnp.float32)
    pre = pre + jnp.dot(h[...], Wr[...], preferred_element_type=jnp.float32)
    for j in range(k):
        cnt = cnts[j][:, 0:1]
        mean = sums[j][...] * (1.0 / jnp.maximum(cnt, 1.0))
        pre = pre + jnp.dot(mean, Wls[j][...], preferred_element_type=jnp.float32)
    pre = pre + bias[...]

    ig = jax.nn.sigmoid(pre[:, 0:_H])
    fg = jax.nn.sigmoid(pre[:, _H:2 * _H])
    tg = jnp.tanh(pre[:, 2 * _H:3 * _H])
    og = jax.nn.sigmoid(pre[:, 3 * _H:4 * _H])
    c0 = fg * c[...] + ig * tg
    h0 = og * jnp.tanh(c0)
    h0o[...] = h0
    c0o[...] = c0

    i = pl.program_id(0)
    hr = jnp.maximum(h0, 0.0)
    row = i * _BLK + lax.broadcasted_iota(jnp.int32, (_BLK, _NB), 0)
    onehot = ((b[...] == lax.broadcasted_iota(jnp.int32, (_BLK, _NB), 1))
              & (row < n)).astype(jnp.float32)
    ps_blk = lax.dot_general(onehot, hr, (((0,), (0,)), ((), ())),
                             preferred_element_type=jnp.float32)
    pc_blk = lax.dot_general(onehot, jnp.ones((_BLK, 8), jnp.float32),
                             (((0,), (0,)), ((), ())),
                             preferred_element_type=jnp.float32)

    @pl.when(i == 0)
    def _():
        pso[...] = jnp.zeros_like(pso)
        pco[...] = jnp.zeros_like(pco)

    pso[...] += ps_blk
    pco[...] += pc_blk


def _dense_call(nt, x, h, c, batch, agg_sums, agg_cnts, Wx, Wr, Wls, bias):
    n = x.shape[0]
    k = len(agg_sums)
    nblk = -(-n // _BLK)
    b2 = batch.reshape(n, 1)

    row = lambda i: (i, 0)
    bcast = lambda i: (0, 0)
    in_specs = (
        [pl.BlockSpec((_BLK, _D), row), pl.BlockSpec((_BLK, _H), row),
         pl.BlockSpec((_BLK, _H), row), pl.BlockSpec((_BLK, 1), row)]
        + [pl.BlockSpec((_BLK, _H), row) for _ in range(k)]
        + [pl.BlockSpec((_BLK, _CNTW), row) for _ in range(k)]
        + [pl.BlockSpec((_D, 4 * _H), bcast), pl.BlockSpec((_H, 4 * _H), bcast)]
        + [pl.BlockSpec((_H, 4 * _H), bcast) for _ in range(k)]
        + [pl.BlockSpec((1, 4 * _H), bcast)]
    )
    out_specs = [
        pl.BlockSpec((_BLK, _H), row), pl.BlockSpec((_BLK, _H), row),
        pl.BlockSpec((_NB, _H), bcast), pl.BlockSpec((_NB, 8), bcast),
    ]
    out_shape = [
        jax.ShapeDtypeStruct((n, _H), jnp.float32),
        jax.ShapeDtypeStruct((n, _H), jnp.float32),
        jax.ShapeDtypeStruct((_NB, _H), jnp.float32),
        jax.ShapeDtypeStruct((_NB, 8), jnp.float32),
    ]
    h0, c0, ps, pc = pl.pallas_call(
        functools.partial(_dense_kernel_body, k, n),
        grid=(nblk,),
        in_specs=in_specs,
        out_specs=out_specs,
        out_shape=out_shape,
    )(x, h, c, b2, *agg_sums, *agg_cnts, Wx, Wr, *Wls, bias)
    return h0, c0, ps, pc


def _final_kernel_body(psa, pca, pst, pct, psu, pcu, Wa, Wt, Wu, b, out):
    acc = b[...]
    for ps, pc, W in ((psa, pca, Wa), (pst, pct, Wt), (psu, pcu, Wu)):
        recip = 1.0 / jnp.maximum(pc[:, 0:1], 1.0)
        acc = acc + jnp.dot(ps[...] * recip, W[...],
                            preferred_element_type=jnp.float32)
    out[...] = acc


def _final_call(pools, lin_W, lin_b):
    Wa, Wt, Wu = lin_W[:_H], lin_W[_H:2 * _H], lin_W[2 * _H:]
    args = []
    for nt in _NT:
        args += [pools[nt][0], pools[nt][1]]
    return pl.pallas_call(
        _final_kernel_body,
        out_shape=jax.ShapeDtypeStruct((_NB, _OUT), jnp.float32),
    )(*args, Wa, Wt, Wu, lin_b.reshape(1, _OUT))


def kernel(x_article, x_tweet, x_user, h_article, h_tweet, h_user,
           c_article, c_tweet, c_user, batch_article, batch_tweet, batch_user,
           edge_index_posts, edge_index_mentions, edge_index_follows,
           edge_index_cites, params):
    x_dict = {"article": x_article, "tweet": x_tweet, "user": x_user}
    h_dict = {"article": h_article, "tweet": h_tweet, "user": h_user}
    c_dict = {"article": c_article, "tweet": c_tweet, "user": c_user}
    batch_dict = {"article": batch_article, "tweet": batch_tweet, "user": batch_user}
    ei_dict = {"posts": edge_index_posts, "mentions": edge_index_mentions,
               "follows": edge_index_follows, "cites": edge_index_cites}

    Wx = {nt: jnp.concatenate([params["W_%s_%s" % (g, nt)] for g in _G], axis=1)
          for nt in _NT}
    Wr = {nt: jnp.concatenate(
        [sum(params["Wr_%s_%s" % (g, r)] for r in _IN_ETS[nt]) for g in _G], axis=1)
        for nt in _NT}
    Wl = {r: jnp.concatenate([params["Wl_%s_%s" % (g, r)] for g in _G], axis=1)
          for r in ei_dict}
    bias = {nt: jnp.concatenate(
        [params["b_%s_%s" % (g, nt)]
         + sum(params["bl_%s_%s" % (g, r)] for r in _IN_ETS[nt])[None, :]
         for g in _G], axis=1)
        for nt in _NT}

    agg = _sc_aggregate(h_tweet, h_user, ei_dict)

    h0, c0, pools = {}, {}, {}
    for nt in _NT:
        rs = _IN_ETS[nt]
        h0[nt], c0[nt], ps, pc = _dense_call(
            nt, x_dict[nt], h_dict[nt], c_dict[nt], batch_dict[nt],
            [agg[r][0] for r in rs], [agg[r][1] for r in rs],
            Wx[nt], Wr[nt], [Wl[r] for r in rs], bias[nt])
        pools[nt] = (ps, pc)

    out = _final_call(pools, params["lin_W"], params["lin_b"])
    return (out, h0["article"], h0["tweet"], h0["user"],
            c0["article"], c0["tweet"], c0["user"])


# speedup vs baseline: 2.2456x; 1.2859x over previous
import functools

import jax
import jax.numpy as jnp
from jax import lax
from jax.experimental import pallas as pl
from jax.experimental.pallas import tpu as pltpu
from jax.experimental.pallas import tpu_sc as plsc

_NT = ("article", "tweet", "user")
_SIZES = {"article": 10000, "tweet": 100000, "user": 50000}
_IN_ETS = {"article": ("cites",), "tweet": ("posts",), "user": ("mentions", "follows")}
_D, _H, _OUT, _NB = 128, 64, 32, 64
_G = ("i", "f", "c", "o")
_BLK = 1024
_CNTW = 16

_NC, _NS = 2, 16
_RND = 256
_SUB = 128
_CMAX = 20480
_DUMP = _CMAX

_ET_SPECS = (
    ("posts", "user", 200000, 49, (20480, 20480, 20480, 20480, 18432), 0),
    ("mentions", "tweet", 150000, 37, (20480, 20480, 9216), 5),
    ("follows", "user", 100000, 25, (20480, 20480, 9216), 8),
    ("cites", "tweet", 150000, 37, (10240,), 11),
)


def _steps(total, maxs):
    out, pos = [], 0
    while pos < total:
        out.append((pos, min(maxs, total - pos)))
        pos += min(maxs, total - pos)
    return out


def _sc_body(h_tweet, h_user, *refs):
    ins = refs[:8]
    outs = refs[8:16]
    (acc_s, acc_c, srcb, dstb, ldst, rows, ones, zc, sem) = refs[16:]
    tables = {"tweet": h_tweet, "user": h_user}

    cid = lax.axis_index("c")
    sid = lax.axis_index("s")

    zvec = jnp.zeros((16,), jnp.float32)
    onevec = jnp.where(lax.iota(jnp.int32, 16) == 0, 1.0, 0.0)

    def zcinit(j, carry):
        zc[j, :] = zvec
        return carry

    lax.fori_loop(0, _SUB, zcinit, 0)

    def oinit(j, carry):
        ones[j, :] = onevec
        return carry

    lax.fori_loop(0, _SUB, oinit, 0)

    dumpvec = _DUMP + lax.iota(jnp.int32, 16)

    for spec_i, (name, src_nt, E, nrnd, chunks, gbase) in enumerate(_ET_SPECS):
        table = tables[src_nt]
        src_ref, dst_ref = ins[2 * spec_i], ins[2 * spec_i + 1]
        sum_ref, cnt_ref = outs[2 * spec_i], outs[2 * spec_i + 1]
        slice_len = nrnd * _RND

        lo = 0
        for ch, C in enumerate(chunks):
            stripe = C // _NS

            @pl.when((gbase + ch) % 2 == cid)
            def _(src_ref=src_ref, dst_ref=dst_ref, sum_ref=sum_ref,
                  cnt_ref=cnt_ref, table=table, nrnd=nrnd, C=C, lo=lo,
                  slice_len=slice_len, stripe=stripe):
                def rz(j, carry):
                    for q in range(_H // 16):
                        rows[j, pl.ds(q * 16, 16)] = zvec
                    return carry

                lax.fori_loop(0, _RND, rz, 0)

                for (zp, zs) in _steps(stripe, _RND):
                    pltpu.sync_copy(rows.at[pl.ds(0, zs)],
                                    acc_s.at[pl.ds(sid * stripe + zp, zs)])
                for (zp, zs) in _steps(stripe, _SUB):
                    pltpu.sync_copy(zc.at[pl.ds(0, zs)],
                                    acc_c.at[pl.ds(sid * stripe + zp, zs)])

                plsc.subcore_barrier()

                ebase = sid * slice_len

                def rnd(r, carry):
                    eoff = ebase + r * _RND
                    pltpu.sync_copy(src_ref.at[pl.ds(eoff, _RND)], srcb)
                    pltpu.sync_copy(dst_ref.at[pl.ds(eoff, _RND)], dstb)

                    def vr(v, c2):
                        dv = dstb[pl.ds(v * 16, 16)]
                        m = (dv >= lo) & (dv < lo + C)
                        ldst[pl.ds(v * 16, 16)] = jnp.where(m, dv - lo, dumpvec)
                        return c2

                    lax.fori_loop(0, _RND // 16, vr, 0)

                    cps = [pltpu.async_copy(
                        table.at[srcb.at[pl.ds(j * _SUB, _SUB)]],
                        rows.at[pl.ds(j * _SUB, _SUB)], sem)
                        for j in range(_RND // _SUB)]
                    for cp in cps:
                        cp.wait()
                    cps = []
                    for j in range(_RND // _SUB):
                        cps.append(pltpu.async_copy(
                            rows.at[pl.ds(j * _SUB, _SUB)],
                            acc_s.at[ldst.at[pl.ds(j * _SUB, _SUB)]],
                            sem, add=True))
                        cps.append(pltpu.async_copy(
                            ones,
                            acc_c.at[ldst.at[pl.ds(j * _SUB, _SUB)]],
                            sem, add=True))
                    for cp in cps:
                        cp.wait()
                    return carry

                lax.fori_loop(0, nrnd, rnd, 0)

                plsc.subcore_barrier()

                for (zp, zs) in _steps(stripe, _RND):
                    pltpu.sync_copy(acc_s.at[pl.ds(sid * stripe + zp, zs)],
                                    sum_ref.at[pl.ds(lo + sid * stripe + zp, zs)])
                for (zp, zs) in _steps(stripe, _SUB):
                    pltpu.sync_copy(acc_c.at[pl.ds(sid * stripe + zp, zs)],
                                    cnt_ref.at[pl.ds(lo + sid * stripe + zp, zs)])

            lo += C


def _sc_aggregate(h_tweet, h_user, ei_dict):
    args = []
    for (name, src_nt, E, nrnd, chunks, gbase) in _ET_SPECS:
        ei = ei_dict[name]
        e_pad = _NS * nrnd * _RND
        pe = e_pad - ei.shape[1]
        args.append(jnp.pad(ei[0], (0, pe)))
        args.append(jnp.pad(ei[1], (0, pe), constant_values=-1))

    out_type = []
    for (name, src_nt, E, nrnd, chunks, gbase) in _ET_SPECS:
        nd_pad = sum(chunks)
        out_type.append(jax.ShapeDtypeStruct((nd_pad, _H), jnp.float32))
        out_type.append(jax.ShapeDtypeStruct((nd_pad, _CNTW), jnp.float32))

    scratch = [
        pltpu.VMEM_SHARED((_CMAX + 16, _H), jnp.float32),
        pltpu.VMEM_SHARED((_CMAX + 16, _CNTW), jnp.float32),
        pltpu.VMEM((_RND,), jnp.int32),
        pltpu.VMEM((_RND,), jnp.int32),
        pltpu.VMEM((_RND,), jnp.int32),
        pltpu.VMEM((_RND, _H), jnp.float32),
        pltpu.VMEM((_SUB, _CNTW), jnp.float32),
        pltpu.VMEM((_SUB, _CNTW), jnp.float32),
        pltpu.SemaphoreType.DMA,
    ]
    mesh = plsc.VectorSubcoreMesh(core_axis_name="c", subcore_axis_name="s")
    call = pl.kernel(_sc_body, out_type=out_type, mesh=mesh,
                     scratch_types=scratch,
                     compiler_params=pltpu.CompilerParams(
                         use_tc_tiling_on_sc=False))
    outs = call(h_tweet, h_user, *args)
    res = {}
    for i, (name, *_rest) in enumerate(_ET_SPECS):
        res[name] = (outs[2 * i], outs[2 * i + 1])
    return res


def _pre_kernel_body(x, h, Wx, Wr, bias, preo):
    pre = jnp.dot(x[...], Wx[...], preferred_element_type=jnp.float32)
    pre = pre + jnp.dot(h[...], Wr[...], preferred_element_type=jnp.float32)
    preo[...] = pre + bias[...]


def _pre_call(x, h, Wx, Wr, bias):
    n = x.shape[0]
    nblk = -(-n // _BLK)
    row = lambda i: (i, 0)
    bcast = lambda i: (0, 0)
    return pl.pallas_call(
        _pre_kernel_body,
        grid=(nblk,),
        in_specs=[pl.BlockSpec((_BLK, _D), row), pl.BlockSpec((_BLK, _H), row),
                  pl.BlockSpec((_D, 4 * _H), bcast),
                  pl.BlockSpec((_H, 4 * _H), bcast),
                  pl.BlockSpec((1, 4 * _H), bcast)],
        out_specs=pl.BlockSpec((_BLK, 4 * _H), row),
        out_shape=jax.ShapeDtypeStruct((n, 4 * _H), jnp.float32),
    )(x, h, Wx, Wr, bias)


def _dense_kernel_body(k, n, *refs):
    pre0, c, b = refs[0], refs[1], refs[2]
    sums = refs[3:3 + k]
    cnts = refs[3 + k:3 + 2 * k]
    Wls = refs[3 + 2 * k:3 + 3 * k]
    h0o, c0o, pso, pco = refs[3 + 3 * k:7 + 3 * k]

    pre = pre0[...]
    for j in range(k):
        cnt = cnts[j][:, 0:1]
        mean = sums[j][...] * (1.0 / jnp.maximum(cnt, 1.0))
        pre = pre + jnp.dot(mean, Wls[j][...], preferred_element_type=jnp.float32)

    ig = jax.nn.sigmoid(pre[:, 0:_H])
    fg = jax.nn.sigmoid(pre[:, _H:2 * _H])
    tg = jnp.tanh(pre[:, 2 * _H:3 * _H])
    og = jax.nn.sigmoid(pre[:, 3 * _H:4 * _H])
    c0 = fg * c[...] + ig * tg
    h0 = og * jnp.tanh(c0)
    h0o[...] = h0
    c0o[...] = c0

    i = pl.program_id(0)
    hr = jnp.maximum(h0, 0.0)
    row = i * _BLK + lax.broadcasted_iota(jnp.int32, (_BLK, _NB), 0)
    onehot = ((b[...] == lax.broadcasted_iota(jnp.int32, (_BLK, _NB), 1))
              & (row < n)).astype(jnp.float32)
    ps_blk = lax.dot_general(onehot, hr, (((0,), (0,)), ((), ())),
                             preferred_element_type=jnp.float32)
    pc_blk = lax.dot_general(onehot, jnp.ones((_BLK, 8), jnp.float32),
                             (((0,), (0,)), ((), ())),
                             preferred_element_type=jnp.float32)

    @pl.when(i == 0)
    def _():
        pso[...] = jnp.zeros_like(pso)
        pco[...] = jnp.zeros_like(pco)

    pso[...] += ps_blk
    pco[...] += pc_blk


def _dense_call(nt, pre0, c, batch, agg_sums, agg_cnts, Wls):
    n = pre0.shape[0]
    k = len(agg_sums)
    nblk = -(-n // _BLK)
    b2 = batch.reshape(n, 1)

    row = lambda i: (i, 0)
    bcast = lambda i: (0, 0)
    in_specs = (
        [pl.BlockSpec((_BLK, 4 * _H), row), pl.BlockSpec((_BLK, _H), row),
         pl.BlockSpec((_BLK, 1), row)]
        + [pl.BlockSpec((_BLK, _H), row) for _ in range(k)]
        + [pl.BlockSpec((_BLK, _CNTW), row) for _ in range(k)]
        + [pl.BlockSpec((_H, 4 * _H), bcast) for _ in range(k)]
    )
    out_specs = [
        pl.BlockSpec((_BLK, _H), row), pl.BlockSpec((_BLK, _H), row),
        pl.BlockSpec((_NB, _H), bcast), pl.BlockSpec((_NB, 8), bcast),
    ]
    out_shape = [
        jax.ShapeDtypeStruct((n, _H), jnp.float32),
        jax.ShapeDtypeStruct((n, _H), jnp.float32),
        jax.ShapeDtypeStruct((_NB, _H), jnp.float32),
        jax.ShapeDtypeStruct((_NB, 8), jnp.float32),
    ]
    h0, c0, ps, pc = pl.pallas_call(
        functools.partial(_dense_kernel_body, k, n),
        grid=(nblk,),
        in_specs=in_specs,
        out_specs=out_specs,
        out_shape=out_shape,
    )(pre0, c, b2, *agg_sums, *agg_cnts, *Wls)
    return h0, c0, ps, pc


def _final_kernel_body(psa, pca, pst, pct, psu, pcu, Wa, Wt, Wu, b, out):
    acc = b[...]
    for ps, pc, W in ((psa, pca, Wa), (pst, pct, Wt), (psu, pcu, Wu)):
        recip = 1.0 / jnp.maximum(pc[:, 0:1], 1.0)
        acc = acc + jnp.dot(ps[...] * recip, W[...],
                            preferred_element_type=jnp.float32)
    out[...] = acc


def _final_call(pools, lin_W, lin_b):
    Wa, Wt, Wu = lin_W[:_H], lin_W[_H:2 * _H], lin_W[2 * _H:]
    args = []
    for nt in _NT:
        args += [pools[nt][0], pools[nt][1]]
    return pl.pallas_call(
        _final_kernel_body,
        out_shape=jax.ShapeDtypeStruct((_NB, _OUT), jnp.float32),
    )(*args, Wa, Wt, Wu, lin_b.reshape(1, _OUT))


def kernel(x_article, x_tweet, x_user, h_article, h_tweet, h_user,
           c_article, c_tweet, c_user, batch_article, batch_tweet, batch_user,
           edge_index_posts, edge_index_mentions, edge_index_follows,
           edge_index_cites, params):
    x_dict = {"article": x_article, "tweet": x_tweet, "user": x_user}
    h_dict = {"article": h_article, "tweet": h_tweet, "user": h_user}
    c_dict = {"article": c_article, "tweet": c_tweet, "user": c_user}
    batch_dict = {"article": batch_article, "tweet": batch_tweet, "user": batch_user}
    ei_dict = {"posts": edge_index_posts, "mentions": edge_index_mentions,
               "follows": edge_index_follows, "cites": edge_index_cites}

    Wx = {nt: jnp.concatenate([params["W_%s_%s" % (g, nt)] for g in _G], axis=1)
          for nt in _NT}
    Wr = {nt: jnp.concatenate(
        [sum(params["Wr_%s_%s" % (g, r)] for r in _IN_ETS[nt]) for g in _G], axis=1)
        for nt in _NT}
    Wl = {r: jnp.concatenate([params["Wl_%s_%s" % (g, r)] for g in _G], axis=1)
          for r in ei_dict}
    bias = {nt: jnp.concatenate(
        [params["b_%s_%s" % (g, nt)]
         + sum(params["bl_%s_%s" % (g, r)] for r in _IN_ETS[nt])[None, :]
         for g in _G], axis=1)
        for nt in _NT}

    agg = _sc_aggregate(h_tweet, h_user, ei_dict)

    pre0 = {nt: _pre_call(x_dict[nt], h_dict[nt], Wx[nt], Wr[nt], bias[nt])
            for nt in _NT}

    h0, c0, pools = {}, {}, {}
    for nt in _NT:
        rs = _IN_ETS[nt]
        h0[nt], c0[nt], ps, pc = _dense_call(
            nt, pre0[nt], c_dict[nt], batch_dict[nt],
            [agg[r][0] for r in rs], [agg[r][1] for r in rs],
            [Wl[r] for r in rs])
        pools[nt] = (ps, pc)

    out = _final_call(pools, params["lin_W"], params["lin_b"])
    return (out, h0["article"], h0["tweet"], h0["user"],
            c0["article"], c0["tweet"], c0["user"])
